# baseline (speedup 1.0000x reference)
import dataclasses
import functools

import jax
import jax.numpy as jnp
from jax import lax
from jax.experimental import pallas as pl
from jax.experimental.pallas import tpu as pltpu
from jax.experimental.pallas import tpu_sc as plsc

def _sc_compiler_params():
    cp = pltpu.CompilerParams()
    if "needs_layout_passes" in pltpu.CompilerParams.__dataclass_fields__:
        cp = dataclasses.replace(cp, needs_layout_passes=False)
    return cp


_T = 1024
_FT = 512
_L = 16
_NW = 32
_RC = 16


def _count_masked(xc_ref, lo, hi):
    def body(j, acc):
        v = xc_ref[pl.ds(j * _L, _L)]
        return acc + plsc.all_reduce_population_count(v <= 0.0)
    return lax.fori_loop(lo, hi, body, jnp.zeros((_L,), jnp.int32))


def _route_body(xcol_hbm, x_hbm, invp_hbm, counts_hbm, xs_hbm,
                xc_ref, idx_ref, rows_ref, rows2_ref, cnt_ref,
                si0, si1, so0, so1):
    sem_i, sem_o = [si0, si1], [so0, so1]
    wid = lax.axis_index("s") * 2 + lax.axis_index("c")
    n = xcol_hbm.shape[0]
    chunk = n // _NW
    nv = chunk // _L
    base = wid * chunk

    pltpu.sync_copy(xcol_hbm, xc_ref)

    nc = chunk // _RC
    rows = [rows_ref, rows2_ref]
    in_h = [None] * nc
    out_h = [None] * nc
    in_h[0] = pltpu.async_copy(x_hbm.at[pl.ds(base, _RC)], rows[0], sem_i[0])
    in_h[1] = pltpu.async_copy(x_hbm.at[pl.ds(base + _RC, _RC)], rows[1],
                               sem_i[1])

    pre = _count_masked(xc_ref, 0, wid * nv)
    c1 = pre + _count_masked(xc_ref, wid * nv, n // _L)
    s2 = (c1 + (_T - 1)) & (-_T)

    @pl.when(wid == 0)
    def _():
        cnt_ref[:] = c1
        pltpu.sync_copy(cnt_ref, counts_hbm)

    iota = lax.iota(jnp.int32, _L)
    n1 = pre
    for c in range(nc):
        b = c & 1
        v = xc_ref[pl.ds((wid * nv + c) * _L, _L)]
        m = v <= 0.0
        mi = jnp.where(m, 1, 0)
        excl1 = n1 + plsc.cumsum(mi) - mi
        pos = base + c * _L + iota
        idx_ref[c, :] = jnp.where(m, excl1, s2 + (pos - excl1))
        n1 = n1 + plsc.all_reduce_population_count(m)
        in_h[c].wait()
        out_h[c] = pltpu.async_copy(rows[b], xs_hbm.at[idx_ref.at[c]],
                                    sem_o[b])
        if 1 <= c < nc - 1:
            out_h[c - 1].wait()
            in_h[c + 1] = pltpu.async_copy(
                x_hbm.at[pl.ds(base + (c + 1) * _RC, _RC)], rows[1 - b],
                sem_i[1 - b])
    out_h[nc - 2].wait()
    out_h[nc - 1].wait()

    pltpu.sync_copy(idx_ref, invp_hbm.at[wid])


def _unpermute_body(outs_hbm, invp_hbm, out_hbm, idx_ref, rows_ref,
                    rows2_ref, si0, si1, so0, so1):
    sem_i, sem_o = [si0, si1], [so0, so1]
    wid = lax.axis_index("s") * 2 + lax.axis_index("c")
    n = out_hbm.shape[0]
    chunk = n // _NW
    base = wid * chunk
    pltpu.sync_copy(invp_hbm.at[wid], idx_ref)
    nc = chunk // _RC
    rows = [rows_ref, rows2_ref]
    in_h = [None] * nc
    out_h = [None] * nc
    in_h[0] = pltpu.async_copy(outs_hbm.at[idx_ref.at[0]], rows[0], sem_i[0])
    for c in range(nc):
        b = c & 1
        in_h[c].wait()
        out_h[c] = pltpu.async_copy(rows[b],
                                    out_hbm.at[pl.ds(base + c * _RC, _RC)],
                                    sem_o[b])
        if c + 1 < nc:
            if c >= 1:
                out_h[c - 1].wait()
            in_h[c + 1] = pltpu.async_copy(outs_hbm.at[idx_ref.at[c + 1]],
                                           rows[1 - b], sem_i[1 - b])
    out_h[nc - 2].wait()
    out_h[nc - 1].wait()


def _moe_body(em_ref, x_ref, wa1_ref, ba1_ref, wb1_ref, bb1_ref,
              wa2_ref, ba2_ref, wb2_ref, bb2_ref, out_ref, *, nf):
    t, f = pl.program_id(0), pl.program_id(1)

    def expert(wa_ref, ba_ref, wb_ref, bb_ref):
        @pl.when(f == 0)
        def _init():
            out_ref[:] = jnp.broadcast_to(bb_ref[0:1, :], out_ref.shape)

        h = jnp.dot(x_ref[:], wa_ref[:], preferred_element_type=jnp.float32)
        h = jnp.maximum(h + ba_ref[0:1, :], 0.0).astype(jnp.bfloat16)
        out_ref[:] += jnp.dot(h, wb_ref[:], preferred_element_type=jnp.float32)

    @pl.when(em_ref[t] == 0)
    def _e1():
        expert(wa1_ref, ba1_ref, wb1_ref, bb1_ref)

    @pl.when(em_ref[t] != 0)
    def _e2():
        expert(wa2_ref, ba2_ref, wb2_ref, bb2_ref)


def kernel(x, W1a, b1a, W1b, b1b, W2a, b2a, W2b, b2b):
    n, d = x.shape
    f_dim = W1a.shape[1]
    np_ = n + _T
    nt, nf = np_ // _T, f_dim // _FT
    chunk = n // _NW
    mesh = plsc.VectorSubcoreMesh(core_axis_name="c", subcore_axis_name="s")

    route = pl.kernel(
        _route_body,
        mesh=mesh,
        out_type=[
            jax.ShapeDtypeStruct((_NW, chunk // _RC, _RC), jnp.int32),
            jax.ShapeDtypeStruct((_L,), jnp.int32),
            jax.ShapeDtypeStruct((np_, d), jnp.float32),
        ],
        scratch_types=[
            pltpu.VMEM((n,), jnp.float32),
            pltpu.VMEM((chunk // _RC, _RC), jnp.int32),
            pltpu.VMEM((_RC, d), jnp.float32),
            pltpu.VMEM((_RC, d), jnp.float32),
            pltpu.VMEM((_L,), jnp.int32),
            pltpu.SemaphoreType.DMA,
            pltpu.SemaphoreType.DMA,
            pltpu.SemaphoreType.DMA,
            pltpu.SemaphoreType.DMA,
        ],
        compiler_params=_sc_compiler_params(),
    )
    inv_perm, counts, x_s = route(x[:, 0], x)

    c1 = counts[0]
    nt1 = (c1 + _T - 1) // _T
    em = (jnp.arange(nt, dtype=jnp.int32) >= nt1).astype(jnp.int32)

    def wsel(e):
        def fa(i, j, em):
            return (0, jnp.where(em[i] == e, j, 0))

        def fb(i, j, em):
            return (jnp.where(em[i] == e, j, 0), 0)

        def fbias(i, j, em):
            return (0, jnp.where(em[i] == e, j, 0))

        return (
            pl.BlockSpec((d, _FT), fa),
            pl.BlockSpec((1, _FT), fbias),
            pl.BlockSpec((_FT, d), fb),
            pl.BlockSpec((1, d), lambda i, j, em: (0, 0)),
        )

    out_s = pl.pallas_call(
        functools.partial(_moe_body, nf=nf),
        grid_spec=pltpu.PrefetchScalarGridSpec(
            num_scalar_prefetch=1,
            grid=(nt, nf),
            in_specs=[
                pl.BlockSpec((_T, d), lambda i, j, em: (i, 0)),
                *wsel(0), *wsel(1),
            ],
            out_specs=pl.BlockSpec((_T, d), lambda i, j, em: (i, 0)),
        ),
        out_shape=jax.ShapeDtypeStruct((np_, d), jnp.float32),
        compiler_params=pltpu.CompilerParams(
            dimension_semantics=("parallel", "arbitrary"),
        ),
    )(em, x_s,
      W1a, b1a[None, :], W1b.astype(jnp.bfloat16), b1b[None, :],
      W2a, b2a[None, :], W2b.astype(jnp.bfloat16), b2b[None, :])

    unpermute = pl.kernel(
        _unpermute_body,
        mesh=mesh,
        out_type=jax.ShapeDtypeStruct((n, d), jnp.float32),
        scratch_types=[
            pltpu.VMEM((chunk // _RC, _RC), jnp.int32),
            pltpu.VMEM((_RC, d), jnp.float32),
            pltpu.VMEM((_RC, d), jnp.float32),
            pltpu.SemaphoreType.DMA,
            pltpu.SemaphoreType.DMA,
            pltpu.SemaphoreType.DMA,
            pltpu.SemaphoreType.DMA,
        ],
        compiler_params=_sc_compiler_params(),
    )
    return unpermute(out_s, inv_perm)
